# baseline (speedup 1.0000x reference)
import functools

import jax
import jax.numpy as jnp
from jax import lax
from jax.experimental import pallas as pl
from jax.experimental.pallas import tpu as pltpu
from jax.experimental.pallas import tpu_sc as plsc

N = 10000
E = 320000
D = 128
H = 64
G = 128

NC = 2
NS = 16
NW = NC * NS
N_PAD = 10240
ROWS_PER_SUB = N_PAD // NS
CH = 128
C0 = 44
C1 = 116
CMAX = max(C0, C1)
E_PAD = NS * (C0 + C1) * CH
RB = 1280
GRID = N_PAD // RB


@functools.cache
def _make_sc_pass(h):
    mesh = plsc.VectorSubcoreMesh(core_axis_name="c", subcore_axis_name="s")

    @functools.partial(
        pl.kernel,
        out_type=jax.ShapeDtypeStruct((NC, N_PAD, h), jnp.float32),
        mesh=mesh,
        scratch_types=[
            pltpu.VMEM((CMAX, CH), jnp.int32),
            pltpu.VMEM((CMAX, CH), jnp.int32),
            pltpu.VMEM((CH, h), jnp.float32),
            pltpu.VMEM((CH, h), jnp.float32),
            pltpu.VMEM_SHARED((N_PAD, h), jnp.float32),
            pltpu.SemaphoreType.DMA,
            pltpu.SemaphoreType.DMA,
        ],
        compiler_params=pltpu.CompilerParams(use_tc_tiling_on_sc=False),
    )
    def sc_pass(table, src_g, dst_g, out, src_v, dst_v, buf0, buf1, acc,
                sem0, sem1):
        cid = lax.axis_index("c")
        sid = lax.axis_index("s")
        wid = sid * NC + cid
        r0 = sid * ROWS_PER_SUB
        nch2 = jnp.where(cid == 0, C0 // 2, C1 // 2)
        pltpu.sync_copy(table.at[pl.ds(r0, ROWS_PER_SUB)],
                        acc.at[pl.ds(r0, ROWS_PER_SUB)])
        pltpu.sync_copy(src_g.at[wid], src_v)
        pltpu.sync_copy(dst_g.at[wid], dst_v)
        plsc.subcore_barrier()

        pltpu.async_copy(table.at[src_v.at[0]], buf0, sem0)

        def body(j, carry):
            i = j * 2
            pltpu.async_copy(table.at[src_v.at[i + 1]], buf1, sem1)
            pltpu.make_async_copy(table.at[src_v.at[0]], buf0, sem0).wait()
            pltpu.sync_copy(buf0, acc.at[dst_v.at[i]], add=True)

            @pl.when(j < nch2 - 1)
            def _():
                pltpu.async_copy(table.at[src_v.at[i + 2]], buf0, sem0)

            pltpu.make_async_copy(table.at[src_v.at[0]], buf1, sem1).wait()
            pltpu.sync_copy(buf1, acc.at[dst_v.at[i + 1]], add=True)
            return carry

        lax.fori_loop(0, nch2, body, 0)
        plsc.subcore_barrier()
        pltpu.sync_copy(acc.at[pl.ds(r0, ROWS_PER_SUB)],
                        out.at[cid, pl.ds(r0, ROWS_PER_SUB)])

    return sc_pass


@functools.cache
def _make_sc_deg():
    h = 16
    mesh = plsc.VectorSubcoreMesh(core_axis_name="c", subcore_axis_name="s")

    @functools.partial(
        pl.kernel,
        out_type=jax.ShapeDtypeStruct((NC, N_PAD, h), jnp.float32),
        mesh=mesh,
        scratch_types=[
            pltpu.VMEM((CMAX, CH), jnp.int32),
            pltpu.VMEM((CH, h), jnp.float32),
            pltpu.VMEM_SHARED((N_PAD, h), jnp.float32),
        ],
        compiler_params=pltpu.CompilerParams(use_tc_tiling_on_sc=False),
    )
    def sc_deg(ones16, dst_g, out, dst_v, ones_v, acc):
        cid = lax.axis_index("c")
        sid = lax.axis_index("s")
        wid = sid * NC + cid
        r0 = sid * ROWS_PER_SUB
        nch = jnp.where(cid == 0, C0, C1)
        pltpu.sync_copy(ones16.at[pl.ds(r0, ROWS_PER_SUB)],
                        acc.at[pl.ds(r0, ROWS_PER_SUB)])
        pltpu.sync_copy(ones16.at[pl.ds(0, CH)], ones_v)
        pltpu.sync_copy(dst_g.at[wid], dst_v)
        plsc.subcore_barrier()

        def body(i, carry):
            pltpu.sync_copy(ones_v, acc.at[dst_v.at[i]], add=True)
            return carry

        lax.fori_loop(0, nch, body, 0)
        plsc.subcore_barrier()
        pltpu.sync_copy(acc.at[pl.ds(r0, ROWS_PER_SUB)],
                        out.at[cid, pl.ds(r0, ROWS_PER_SUB)])

    return sc_deg


def _sc_pass64(table, src_g, dst_g):
    return _make_sc_pass(H)(table, src_g, dst_g)


def _tc_first(dp0, dp1, x, w1):
    def body(dp0_r, dp1_r, x_r, w1_r, dinv_r, hws_r):
        degt = dp0_r[...][:, 0:1] + dp1_r[...][:, 0:1] - 1.0
        r = lax.rsqrt(degt)
        dinv = r * (1.5 - 0.5 * degt * r * r)
        dinv_r[...] = dinv
        hws_r[...] = dinv * jnp.dot(x_r[...], w1_r[...],
                                    preferred_element_type=jnp.float32)

    return pl.pallas_call(
        body,
        grid=(GRID,),
        in_specs=[
            pl.BlockSpec((RB, 16), lambda i: (i, 0)),
            pl.BlockSpec((RB, 16), lambda i: (i, 0)),
            pl.BlockSpec((RB, D), lambda i: (i, 0)),
            pl.BlockSpec((D, H), lambda i: (0, 0)),
        ],
        out_specs=[
            pl.BlockSpec((RB, 1), lambda i: (i, 0)),
            pl.BlockSpec((RB, H), lambda i: (i, 0)),
        ],
        out_shape=[
            jax.ShapeDtypeStruct((N_PAD, 1), jnp.float32),
            jax.ShapeDtypeStruct((N_PAD, H), jnp.float32),
        ],
    )(dp0, dp1, x, w1)


def _tc_layer(p0, p1, hws, dinv, b, w):
    def body(p0_r, p1_r, hws_r, dinv_r, b_r, w_r, out_r):
        dinv = dinv_r[...]
        agg = p0_r[...] + p1_r[...] - hws_r[...]
        hrelu = jnp.maximum(dinv * agg + b_r[...], 0.0)
        out_r[...] = dinv * jnp.dot(hrelu, w_r[...],
                                    preferred_element_type=jnp.float32)

    return pl.pallas_call(
        body,
        grid=(GRID,),
        in_specs=[
            pl.BlockSpec((RB, H), lambda i: (i, 0)),
            pl.BlockSpec((RB, H), lambda i: (i, 0)),
            pl.BlockSpec((RB, H), lambda i: (i, 0)),
            pl.BlockSpec((RB, 1), lambda i: (i, 0)),
            pl.BlockSpec((1, H), lambda i: (0, 0)),
            pl.BlockSpec((H, H), lambda i: (0, 0)),
        ],
        out_specs=pl.BlockSpec((RB, H), lambda i: (i, 0)),
        out_shape=jax.ShapeDtypeStruct((N_PAD, H), jnp.float32),
    )(p0, p1, hws, dinv, b, w)


def _tc_pool(q0, q1, hws3, dinv, batch2, b3, wl, bl):
    def body(q0_r, q1_r, hws_r, dinv_r, batch_r, b3_r, wl_r, bl_r, out_r,
             sums, cnt):
        i = pl.program_id(0)

        @pl.when(i == 0)
        def _():
            sums[...] = jnp.zeros_like(sums)
            cnt[...] = jnp.zeros_like(cnt)

        h3 = dinv_r[...] * (q0_r[...] + q1_r[...] - hws_r[...]) + b3_r[...]
        mask = (batch_r[...] == lax.broadcasted_iota(jnp.int32, (RB, G), 1)
                ).astype(jnp.float32)
        dn = (((0,), (0,)), ((), ()))
        sums[...] += lax.dot_general(mask, h3, dn,
                                     preferred_element_type=jnp.float32,
                                     precision=lax.Precision.HIGHEST)
        cnt[...] += lax.dot_general(mask, jnp.ones((RB, 1), jnp.float32), dn,
                                    preferred_element_type=jnp.float32,
                                    precision=lax.Precision.HIGHEST)

        @pl.when(i == pl.num_programs(0) - 1)
        def _():
            pooled = sums[...] / jnp.maximum(cnt[...], 1.0)
            out_r[...] = jnp.dot(pooled, wl_r[...],
                                 preferred_element_type=jnp.float32) + bl_r[...]

    return pl.pallas_call(
        body,
        grid=(GRID,),
        in_specs=[
            pl.BlockSpec((RB, H), lambda i: (i, 0)),
            pl.BlockSpec((RB, H), lambda i: (i, 0)),
            pl.BlockSpec((RB, H), lambda i: (i, 0)),
            pl.BlockSpec((RB, 1), lambda i: (i, 0)),
            pl.BlockSpec((RB, 1), lambda i: (i, 0)),
            pl.BlockSpec((1, H), lambda i: (0, 0)),
            pl.BlockSpec((H, 1), lambda i: (0, 0)),
            pl.BlockSpec((1, 1), lambda i: (0, 0)),
        ],
        out_specs=pl.BlockSpec((G, 1), lambda i: (0, 0)),
        out_shape=jax.ShapeDtypeStruct((G, 1), jnp.float32),
        scratch_shapes=[
            pltpu.VMEM((G, H), jnp.float32),
            pltpu.VMEM((G, 1), jnp.float32),
        ],
    )(q0, q1, hws3, dinv, batch2, b3, wl, bl)


def _group_edges(idx):
    pad = jnp.full((E_PAD - E,), N, jnp.int32)
    flat = jnp.concatenate([idx, pad])
    n0 = NS * C0 * CH
    g0 = flat[:n0].reshape(NS, C0, CH)
    g1 = flat[n0:].reshape(NS, C1, CH)
    if C0 > C1:
        g1 = jnp.concatenate(
            [g1, jnp.full((NS, C0 - C1, CH), N, jnp.int32)], axis=1)
    elif C1 > C0:
        g0 = jnp.concatenate(
            [g0, jnp.full((NS, C1 - C0, CH), N, jnp.int32)], axis=1)
    return jnp.stack([g0, g1], axis=1).reshape(NW, CMAX, CH)


def kernel(x, edge_index, edge_attr, batch, W1, b1, W2, b2, W3, b3, Wl, bl):
    src = edge_index[0].astype(jnp.int32)
    dst = edge_index[1].astype(jnp.int32)
    src_g = _group_edges(src)
    dst_g = _group_edges(dst)
    x_pad = jnp.zeros((N_PAD, D), jnp.float32).at[:N].set(x)
    batch2 = jnp.concatenate(
        [batch.astype(jnp.int32), jnp.full((N_PAD - N,), G, jnp.int32)]
    ).reshape(N_PAD, 1)
    ones16 = jnp.ones((N_PAD, 16), jnp.float32)

    dp = _make_sc_deg()(ones16, dst_g)
    dinv, hws1 = _tc_first(dp[0], dp[1], x_pad, W1)
    p1 = _sc_pass64(hws1, src_g, dst_g)
    hws2 = _tc_layer(p1[0], p1[1], hws1, dinv, b1.reshape(1, H), W2)
    p2 = _sc_pass64(hws2, src_g, dst_g)
    hws3 = _tc_layer(p2[0], p2[1], hws2, dinv, b2.reshape(1, H), W3)
    q = _sc_pass64(hws3, src_g, dst_g)
    out = _tc_pool(q[0], q[1], hws3, dinv, batch2, b3.reshape(1, H), Wl,
                   bl.reshape(1, 1))
    return out


# speedup vs baseline: 1.2146x; 1.2146x over previous
import functools

import jax
import jax.numpy as jnp
from jax import lax
from jax.experimental import pallas as pl
from jax.experimental.pallas import tpu as pltpu
from jax.experimental.pallas import tpu_sc as plsc

N = 10000
E = 320000
D = 128
H = 64
G = 128

NC = 2
NS = 16
NW = NC * NS
N_PAD = 10240
ROWS_PER_SUB = N_PAD // NS
CH = 128
C0 = 80
C1 = 80
CMAX = max(C0, C1)
E_PAD = NS * (C0 + C1) * CH
RB = 1280
GRID = N_PAD // RB


@functools.cache
def _make_sc_pass(h):
    mesh = plsc.VectorSubcoreMesh(core_axis_name="c", subcore_axis_name="s")

    @functools.partial(
        pl.kernel,
        out_type=jax.ShapeDtypeStruct((NC, N_PAD, h), jnp.float32),
        mesh=mesh,
        scratch_types=[
            pltpu.VMEM((CMAX, CH), jnp.int32),
            pltpu.VMEM((CMAX, CH), jnp.int32),
            pltpu.VMEM((CH, h), jnp.float32),
            pltpu.VMEM((CH, h), jnp.float32),
            pltpu.VMEM_SHARED((N_PAD, h), jnp.float32),
            pltpu.SemaphoreType.DMA,
            pltpu.SemaphoreType.DMA,
        ],
        compiler_params=pltpu.CompilerParams(use_tc_tiling_on_sc=False),
    )
    def sc_pass(table, src_g, dst_g, out, src_v, dst_v, buf0, buf1, acc,
                sem0, sem1):
        cid = lax.axis_index("c")
        sid = lax.axis_index("s")
        wid = sid * NC + cid
        r0 = sid * ROWS_PER_SUB
        nch2 = jnp.where(cid == 0, C0 // 2, C1 // 2)
        pltpu.sync_copy(table.at[pl.ds(r0, ROWS_PER_SUB)],
                        acc.at[pl.ds(r0, ROWS_PER_SUB)])
        pltpu.sync_copy(src_g.at[wid], src_v)
        pltpu.sync_copy(dst_g.at[wid], dst_v)
        plsc.subcore_barrier()

        pltpu.async_copy(table.at[src_v.at[0]], buf0, sem0)

        def body(j, carry):
            i = j * 2
            pltpu.async_copy(table.at[src_v.at[i + 1]], buf1, sem1)
            pltpu.make_async_copy(table.at[src_v.at[0]], buf0, sem0).wait()
            pltpu.sync_copy(buf0, acc.at[dst_v.at[i]], add=True)

            @pl.when(j < nch2 - 1)
            def _():
                pltpu.async_copy(table.at[src_v.at[i + 2]], buf0, sem0)

            pltpu.make_async_copy(table.at[src_v.at[0]], buf1, sem1).wait()
            pltpu.sync_copy(buf1, acc.at[dst_v.at[i + 1]], add=True)
            return carry

        lax.fori_loop(0, nch2, body, 0)
        plsc.subcore_barrier()
        pltpu.sync_copy(acc.at[pl.ds(r0, ROWS_PER_SUB)],
                        out.at[cid, pl.ds(r0, ROWS_PER_SUB)])

    return sc_pass


@functools.cache
def _make_sc_deg():
    h = 16
    mesh = plsc.VectorSubcoreMesh(core_axis_name="c", subcore_axis_name="s")

    @functools.partial(
        pl.kernel,
        out_type=jax.ShapeDtypeStruct((NC, N_PAD, h), jnp.float32),
        mesh=mesh,
        scratch_types=[
            pltpu.VMEM((CMAX, CH), jnp.int32),
            pltpu.VMEM((CH, h), jnp.float32),
            pltpu.VMEM_SHARED((N_PAD, h), jnp.float32),
        ],
        compiler_params=pltpu.CompilerParams(use_tc_tiling_on_sc=False),
    )
    def sc_deg(ones16, dst_g, out, dst_v, ones_v, acc):
        cid = lax.axis_index("c")
        sid = lax.axis_index("s")
        wid = sid * NC + cid
        r0 = sid * ROWS_PER_SUB
        nch = jnp.where(cid == 0, C0, C1)
        pltpu.sync_copy(ones16.at[pl.ds(r0, ROWS_PER_SUB)],
                        acc.at[pl.ds(r0, ROWS_PER_SUB)])
        pltpu.sync_copy(ones16.at[pl.ds(0, CH)], ones_v)
        pltpu.sync_copy(dst_g.at[wid], dst_v)
        plsc.subcore_barrier()

        def body(i, carry):
            pltpu.sync_copy(ones_v, acc.at[dst_v.at[i]], add=True)
            return carry

        lax.fori_loop(0, nch, body, 0)
        plsc.subcore_barrier()
        pltpu.sync_copy(acc.at[pl.ds(r0, ROWS_PER_SUB)],
                        out.at[cid, pl.ds(r0, ROWS_PER_SUB)])

    return sc_deg


def _sc_pass64(table, src_g, dst_g):
    return _make_sc_pass(H)(table, src_g, dst_g)


def _tc_first(dp0, dp1, x, w1):
    def body(dp0_r, dp1_r, x_r, w1_r, dinv_r, hws_r):
        degt = dp0_r[...][:, 0:1] + dp1_r[...][:, 0:1] - 1.0
        r = lax.rsqrt(degt)
        dinv = r * (1.5 - 0.5 * degt * r * r)
        dinv_r[...] = dinv
        hws_r[...] = dinv * jnp.dot(x_r[...], w1_r[...],
                                    preferred_element_type=jnp.float32)

    return pl.pallas_call(
        body,
        grid=(GRID,),
        in_specs=[
            pl.BlockSpec((RB, 16), lambda i: (i, 0)),
            pl.BlockSpec((RB, 16), lambda i: (i, 0)),
            pl.BlockSpec((RB, D), lambda i: (i, 0)),
            pl.BlockSpec((D, H), lambda i: (0, 0)),
        ],
        out_specs=[
            pl.BlockSpec((RB, 1), lambda i: (i, 0)),
            pl.BlockSpec((RB, H), lambda i: (i, 0)),
        ],
        out_shape=[
            jax.ShapeDtypeStruct((N_PAD, 1), jnp.float32),
            jax.ShapeDtypeStruct((N_PAD, H), jnp.float32),
        ],
    )(dp0, dp1, x, w1)


def _tc_layer(p0, p1, hws, dinv, b, w):
    def body(p0_r, p1_r, hws_r, dinv_r, b_r, w_r, out_r):
        dinv = dinv_r[...]
        agg = p0_r[...] + p1_r[...] - hws_r[...]
        hrelu = jnp.maximum(dinv * agg + b_r[...], 0.0)
        out_r[...] = dinv * jnp.dot(hrelu, w_r[...],
                                    preferred_element_type=jnp.float32)

    return pl.pallas_call(
        body,
        grid=(GRID,),
        in_specs=[
            pl.BlockSpec((RB, H), lambda i: (i, 0)),
            pl.BlockSpec((RB, H), lambda i: (i, 0)),
            pl.BlockSpec((RB, H), lambda i: (i, 0)),
            pl.BlockSpec((RB, 1), lambda i: (i, 0)),
            pl.BlockSpec((1, H), lambda i: (0, 0)),
            pl.BlockSpec((H, H), lambda i: (0, 0)),
        ],
        out_specs=pl.BlockSpec((RB, H), lambda i: (i, 0)),
        out_shape=jax.ShapeDtypeStruct((N_PAD, H), jnp.float32),
    )(p0, p1, hws, dinv, b, w)


def _tc_pool(q0, q1, hws3, dinv, batch2, b3, wl, bl):
    def body(q0_r, q1_r, hws_r, dinv_r, batch_r, b3_r, wl_r, bl_r, out_r,
             sums, cnt):
        i = pl.program_id(0)

        @pl.when(i == 0)
        def _():
            sums[...] = jnp.zeros_like(sums)
            cnt[...] = jnp.zeros_like(cnt)

        h3 = dinv_r[...] * (q0_r[...] + q1_r[...] - hws_r[...]) + b3_r[...]
        mask = (batch_r[...] == lax.broadcasted_iota(jnp.int32, (RB, G), 1)
                ).astype(jnp.float32)
        dn = (((0,), (0,)), ((), ()))
        sums[...] += lax.dot_general(mask, h3, dn,
                                     preferred_element_type=jnp.float32,
                                     precision=lax.Precision.HIGHEST)
        cnt[...] += lax.dot_general(mask, jnp.ones((RB, 1), jnp.float32), dn,
                                    preferred_element_type=jnp.float32,
                                    precision=lax.Precision.HIGHEST)

        @pl.when(i == pl.num_programs(0) - 1)
        def _():
            pooled = sums[...] / jnp.maximum(cnt[...], 1.0)
            out_r[...] = jnp.dot(pooled, wl_r[...],
                                 preferred_element_type=jnp.float32) + bl_r[...]

    return pl.pallas_call(
        body,
        grid=(GRID,),
        in_specs=[
            pl.BlockSpec((RB, H), lambda i: (i, 0)),
            pl.BlockSpec((RB, H), lambda i: (i, 0)),
            pl.BlockSpec((RB, H), lambda i: (i, 0)),
            pl.BlockSpec((RB, 1), lambda i: (i, 0)),
            pl.BlockSpec((RB, 1), lambda i: (i, 0)),
            pl.BlockSpec((1, H), lambda i: (0, 0)),
            pl.BlockSpec((H, 1), lambda i: (0, 0)),
            pl.BlockSpec((1, 1), lambda i: (0, 0)),
        ],
        out_specs=pl.BlockSpec((G, 1), lambda i: (0, 0)),
        out_shape=jax.ShapeDtypeStruct((G, 1), jnp.float32),
        scratch_shapes=[
            pltpu.VMEM((G, H), jnp.float32),
            pltpu.VMEM((G, 1), jnp.float32),
        ],
    )(q0, q1, hws3, dinv, batch2, b3, wl, bl)


def _group_edges(idx):
    pad = jnp.full((E_PAD - E,), N, jnp.int32)
    flat = jnp.concatenate([idx, pad])
    n0 = NS * C0 * CH
    g0 = flat[:n0].reshape(NS, C0, CH)
    g1 = flat[n0:].reshape(NS, C1, CH)
    if C0 > C1:
        g1 = jnp.concatenate(
            [g1, jnp.full((NS, C0 - C1, CH), N, jnp.int32)], axis=1)
    elif C1 > C0:
        g0 = jnp.concatenate(
            [g0, jnp.full((NS, C1 - C0, CH), N, jnp.int32)], axis=1)
    return jnp.stack([g0, g1], axis=1).reshape(NW, CMAX, CH)


def kernel(x, edge_index, edge_attr, batch, W1, b1, W2, b2, W3, b3, Wl, bl):
    src = edge_index[0].astype(jnp.int32)
    dst = edge_index[1].astype(jnp.int32)
    src_g = _group_edges(src)
    dst_g = _group_edges(dst)
    x_pad = jnp.zeros((N_PAD, D), jnp.float32).at[:N].set(x)
    batch2 = jnp.concatenate(
        [batch.astype(jnp.int32), jnp.full((N_PAD - N,), G, jnp.int32)]
    ).reshape(N_PAD, 1)
    ones16 = jnp.ones((N_PAD, 16), jnp.float32)

    dp = _make_sc_deg()(ones16, dst_g)
    dinv, hws1 = _tc_first(dp[0], dp[1], x_pad, W1)
    p1 = _sc_pass64(hws1, src_g, dst_g)
    hws2 = _tc_layer(p1[0], p1[1], hws1, dinv, b1.reshape(1, H), W2)
    p2 = _sc_pass64(hws2, src_g, dst_g)
    hws3 = _tc_layer(p2[0], p2[1], hws2, dinv, b2.reshape(1, H), W3)
    q = _sc_pass64(hws3, src_g, dst_g)
    out = _tc_pool(q[0], q[1], hws3, dinv, batch2, b3.reshape(1, H), Wl,
                   bl.reshape(1, 1))
    return out
